# Initial kernel scaffold; baseline (speedup 1.0000x reference)
import jax
import jax.numpy as jnp
from jax.experimental import pallas as pl


def kernel(x, template_edge_index, emb_feat_a, emb_feat_b, emb_feat_c, emb_feat_d, conv_z_W, conv_z_b, conv_r_W, conv_r_b, conv_h_W, conv_h_b, lin_z_W, lin_z_b, lin_r_W, lin_r_b, lin_h_W, lin_h_b, att, cls_W1, cls_b1, cls_W2, cls_b2):
    raise NotImplementedError("write your pallas kernel here")


# speedup vs baseline: 8.0276x; 8.0276x over previous
import functools

import jax
import jax.numpy as jnp
from jax import lax
from jax.experimental import pallas as pl
from jax.experimental.pallas import tpu as pltpu
from jax.experimental.pallas import tpu_sc as plsc

N = 10000
E = 160000
P = 37
PPAD = 40
NG = 10
GW = 128
HID = 32
VOCAB = 1000
IN_DIM = 32

NC = 2
NS = 16
NW = NC * NS

LK_TOT = P * N * 4
LK_PER_W = 47104
LK_PAD = NW * LK_PER_W
LK_CHUNK = 128
LK_ITERS = LK_PER_W // LK_CHUNK

E_PER_W = E // NW
DEG_CHUNK = 128
DEG_NCHUNKS = E // DEG_CHUNK
DEG_ITERS = -(-DEG_NCHUNKS // NW)
AGG_CHUNK = 128
AGG_NCHUNKS = E // AGG_CHUNK
AGG_ITERS = -(-AGG_NCHUNKS // NW)
ROWS_PER_SUB = N // NS
STRIPE = 624
TAIL = N - NS * STRIPE
TAIL0 = NS * STRIPE
NH = 5
HN = N // NH
HPAD = HN + 8
HSTRIPE = 120
HTAIL = HN - NS * HSTRIPE
HTAIL0 = NS * HSTRIPE

_MESH = dict(core_axis_name="c", subcore_axis_name="s")


def _make_lookup_degree():
    @functools.partial(
        pl.kernel,
        mesh=plsc.VectorSubcoreMesh(**_MESH),
        out_type=[
            jax.ShapeDtypeStruct((LK_PAD, 128), jnp.float32),
            jax.ShapeDtypeStruct((NC, N, 128), jnp.float32),
        ],
        scratch_types=[
            pltpu.VMEM((LK_CHUNK,), jnp.int32),
            pltpu.VMEM((LK_CHUNK, 128), jnp.float32),
            pltpu.VMEM((DEG_CHUNK,), jnp.int32),
            pltpu.VMEM((DEG_CHUNK, 128), jnp.float32),
            pltpu.VMEM_SHARED((N, 128), jnp.float32),
            pltpu.SemaphoreType.DMA,
        ],
    )
    def lookup_degree(table_hbm, idx_hbm, dst_hbm, ones_hbm, zeros_hbm,
                      out_hbm, deg_hbm, idx_v, rows_v, didx_v, ones_v,
                      shared, sem):
        c = lax.axis_index("c")
        s = lax.axis_index("s")
        wid = s * NC + c

        pltpu.sync_copy(zeros_hbm.at[pl.ds(0, STRIPE)],
                        shared.at[pl.ds(s * STRIPE, STRIPE)])

        @pl.when(s == 0)
        def _():
            pltpu.sync_copy(zeros_hbm.at[pl.ds(0, TAIL)],
                            shared.at[pl.ds(TAIL0, TAIL)])

        pltpu.sync_copy(ones_hbm, ones_v)
        plsc.subcore_barrier()

        def dbody(i, _):
            k = i * NW + wid

            @pl.when(k < DEG_NCHUNKS)
            def _():
                base = k * DEG_CHUNK
                pltpu.sync_copy(dst_hbm.at[pl.ds(base, DEG_CHUNK)], didx_v)
                pltpu.sync_copy(ones_v, shared.at[didx_v], add=True)

            return 0

        lax.fori_loop(0, DEG_ITERS, dbody, 0)
        plsc.subcore_barrier()
        pltpu.sync_copy(shared.at[pl.ds(s * STRIPE, STRIPE)],
                        deg_hbm.at[c, pl.ds(s * STRIPE, STRIPE)])

        @pl.when(s == 0)
        def _():
            pltpu.sync_copy(shared.at[pl.ds(TAIL0, TAIL)],
                            deg_hbm.at[c, pl.ds(TAIL0, TAIL)])

        lk_base = wid * LK_PER_W

        def lbody(i, _):
            base = lk_base + i * LK_CHUNK
            pltpu.sync_copy(idx_hbm.at[pl.ds(base, LK_CHUNK)], idx_v)
            pltpu.async_copy(table_hbm.at[idx_v], rows_v, sem).wait()
            pltpu.sync_copy(rows_v, out_hbm.at[pl.ds(base, LK_CHUNK)])
            return 0

        lax.fori_loop(0, LK_ITERS, lbody, 0)

    return lookup_degree


def _make_aggregate():
    @functools.partial(
        pl.kernel,
        mesh=plsc.VectorSubcoreMesh(**_MESH),
        out_type=jax.ShapeDtypeStruct((NC, NG, N, GW), jnp.float32),
        scratch_types=[
            pltpu.VMEM((AGG_CHUNK,), jnp.int32),
            pltpu.VMEM((AGG_CHUNK,), jnp.int32),
            pltpu.VMEM((AGG_CHUNK, GW), jnp.float32),
            pltpu.VMEM_SHARED((HPAD, GW), jnp.float32),
            pltpu.SemaphoreType.DMA,
        ],
    )
    def aggregate(embs_hbm, srcoff_hbm, dsth_hbm, zeros_hbm, out_hbm,
                  src_v, dst_v, rows_v, shared, sem):
        c = lax.axis_index("c")
        s = lax.axis_index("s")
        wid = s * NC + c

        def group(g, _):
            for h in range(NH):
                pltpu.sync_copy(zeros_hbm.at[pl.ds(0, HSTRIPE)],
                                shared.at[pl.ds(s * HSTRIPE, HSTRIPE)])

                @pl.when(s == 0)
                def _():
                    pltpu.sync_copy(zeros_hbm.at[pl.ds(0, HN + 8 - HTAIL0)],
                                    shared.at[pl.ds(HTAIL0, HN + 8 - HTAIL0)])

                plsc.subcore_barrier()

                def body(i, _):
                    k = i * NW + wid

                    @pl.when(k < AGG_NCHUNKS)
                    def _():
                        base = k * AGG_CHUNK
                        pltpu.sync_copy(
                            srcoff_hbm.at[g, 0, pl.ds(base, AGG_CHUNK)], src_v)
                        pltpu.sync_copy(
                            dsth_hbm.at[h, 0, pl.ds(base, AGG_CHUNK)], dst_v)
                        pltpu.async_copy(embs_hbm.at[src_v], rows_v, sem).wait()
                        pltpu.sync_copy(rows_v, shared.at[dst_v], add=True)

                    return 0

                lax.fori_loop(0, AGG_ITERS, body, 0)
                plsc.subcore_barrier()
                pltpu.sync_copy(
                    shared.at[pl.ds(s * HSTRIPE, HSTRIPE)],
                    out_hbm.at[c, g, pl.ds(h * HN + s * HSTRIPE, HSTRIPE)])

                @pl.when(s == 0)
                def _():
                    pltpu.sync_copy(
                        shared.at[pl.ds(HTAIL0, HTAIL)],
                        out_hbm.at[c, g, pl.ds(h * HN + HTAIL0, HTAIL)])

                plsc.subcore_barrier()
            return 0

        lax.fori_loop(0, NG, group, 0)

    return aggregate


def _scale_kernel(emb_ref, degp_ref, out_ref, dinv_ref):
    t = pl.program_id(0)
    q = t % 4
    deg = jnp.sum(degp_ref[:, :, 0], axis=0) + 1.0
    dinv = lax.rsqrt(jnp.maximum(deg, 1.0))
    keep = jnp.where(t < P, 1.0, 0.0).astype(jnp.float32)
    val = emb_ref[0] * (dinv * keep)[:, None]
    tiled = jnp.concatenate([val, val, val, val], axis=-1)
    lane_q = lax.broadcasted_iota(jnp.int32, (1, GW), 1) // IN_DIM
    out_ref[0] = jnp.where(lane_q == q, tiled, out_ref[0])
    dinv_ref[:, 0] = dinv


def _dense_kernel(scat_ref, embs_ref, dinv_ref, att_ref,
                  czW_ref, czb_ref, crW_ref, crb_ref, chW_ref, chb_ref,
                  lzW_ref, lzb_ref, lrW_ref, lrb_ref, lhW_ref, lhb_ref,
                  w1_ref, b1_ref, w2_ref, b2_ref, out_ref):
    f32 = jnp.float32
    dot = lambda a, b: jnp.dot(a, b, preferred_element_type=f32)
    a = att_ref[0, :]
    e = jnp.exp(a - jnp.max(a))
    probs = e / jnp.sum(e)
    lzW = lzW_ref[...]
    lrW = lrW_ref[...]
    lhW = lhW_ref[...]
    Mz = dot(czW_ref[...], lzW[:HID, :])
    Mr = dot(crW_ref[...], lrW[:HID, :])
    Mh = dot(chW_ref[...], lhW[:HID, :])
    bz = dot(czb_ref[...], lzW[:HID, :]) + lzb_ref[...]
    br = dot(crb_ref[...], lrW[:HID, :]) + lrb_ref[...]
    bh = dot(chb_ref[...], lhW[:HID, :]) + lhb_ref[...]
    Uz = lzW[HID:, :]
    Ur = lrW[HID:, :]
    Uh = lhW[HID:, :]
    dinv = dinv_ref[...]

    bn = out_ref.shape[0]
    H = jnp.zeros((bn, HID), f32)
    Hac = jnp.zeros((bn, HID), f32)
    for t in range(P):
        g, q = divmod(t, 4)
        sl = slice(q * HID, (q + 1) * HID)
        agg = scat_ref[0, g, :, sl] + scat_ref[1, g, :, sl] + embs_ref[g, :, sl]
        A = agg * dinv
        Z = jax.nn.sigmoid(dot(A, Mz) + dot(H, Uz) + bz)
        R = jax.nn.sigmoid(dot(A, Mr) + dot(H, Ur) + br)
        Ht = jnp.tanh(dot(A, Mh) + dot(H * R, Uh) + bh)
        H = Z * H + (1.0 - Z) * Ht
        Hac = Hac + probs[t] * H
    h1 = jax.nn.relu(dot(Hac, w1_ref[...]) + b1_ref[...])
    out_ref[...] = dot(h1, w2_ref[...]) + b2_ref[...]


BN = 1000


def kernel(x, template_edge_index, emb_feat_a, emb_feat_b, emb_feat_c,
           emb_feat_d, conv_z_W, conv_z_b, conv_r_W, conv_r_b, conv_h_W,
           conv_h_b, lin_z_W, lin_z_b, lin_r_W, lin_r_b, lin_h_W, lin_h_b,
           att, cls_W1, cls_b1, cls_W2, cls_b2):
    i32 = jnp.int32
    f32 = jnp.float32

    xt = jnp.transpose(x[0], (2, 0, 1)).astype(i32)
    xoff = xt + (jnp.arange(4, dtype=i32) * VOCAB)[None, None, :]
    idx_flat = xoff.reshape(-1)
    idx_pad = jnp.concatenate(
        [idx_flat, jnp.zeros((LK_PAD - LK_TOT,), i32)])
    table = jnp.concatenate(
        [emb_feat_a, emb_feat_b, emb_feat_c, emb_feat_d], axis=0)

    src = template_edge_index[0].astype(i32)
    dst = template_edge_index[1].astype(i32)
    src_off = (src[None, :] + (jnp.arange(NG, dtype=i32) * N)[:, None]
               ).reshape(NG, 1, E)
    h0 = (jnp.arange(NH, dtype=i32) * HN)[:, None]
    in_half = (dst[None, :] >= h0) & (dst[None, :] < h0 + HN)
    dst_half = jnp.where(in_half, dst[None, :] - h0, HN).reshape(NH, 1, E)

    zeros_rows = jnp.zeros((ROWS_PER_SUB, GW), f32)
    ones8 = jnp.ones((DEG_CHUNK, 128), f32)
    zeros8 = zeros_rows
    table_pad = jnp.pad(table, ((0, 0), (0, 128 - 8)))

    rows, degp = _make_lookup_degree()(table_pad, idx_pad, dst, ones8, zeros8)
    emb = rows[:LK_TOT, :8].reshape(P, N, 4 * 8)

    embs_g, dinv = pl.pallas_call(
        _scale_kernel,
        grid=(PPAD,),
        in_specs=[
            pl.BlockSpec((1, N, IN_DIM), lambda t: (jnp.minimum(t, P - 1), 0, 0)),
            pl.BlockSpec((NC, N, 128), lambda t: (0, 0, 0)),
        ],
        out_specs=[
            pl.BlockSpec((1, N, GW), lambda t: (t // 4, 0, 0)),
            pl.BlockSpec((N, 1), lambda t: (0, 0)),
        ],
        out_shape=[
            jax.ShapeDtypeStruct((NG, N, GW), f32),
            jax.ShapeDtypeStruct((N, 1), f32),
        ],
    )(emb, degp)

    embs_flat = embs_g.reshape(NG * N, GW)

    scat = _make_aggregate()(embs_flat, src_off, dst_half, zeros_rows)

    biases = [b.reshape(1, -1) for b in
              (conv_z_b, conv_r_b, conv_h_b, lin_z_b, lin_r_b, lin_h_b,
               cls_b1, cls_b2)]
    czb, crb, chb, lzb, lrb, lhb, b1, b2 = biases
    full = lambda shape: pl.BlockSpec(shape, lambda nb: tuple(0 for _ in shape))
    out = pl.pallas_call(
        _dense_kernel,
        grid=(N // BN,),
        in_specs=[
            pl.BlockSpec((NC, NG, BN, GW), lambda nb: (0, 0, nb, 0)),
            pl.BlockSpec((NG, BN, GW), lambda nb: (0, nb, 0)),
            pl.BlockSpec((BN, 1), lambda nb: (nb, 0)),
            full((1, P)),
            full((IN_DIM, HID)), full((1, HID)),
            full((IN_DIM, HID)), full((1, HID)),
            full((IN_DIM, HID)), full((1, HID)),
            full((2 * HID, HID)), full((1, HID)),
            full((2 * HID, HID)), full((1, HID)),
            full((2 * HID, HID)), full((1, HID)),
            full((HID, HID)), full((1, HID)),
            full((HID, 2)), full((1, 2)),
        ],
        out_specs=pl.BlockSpec((BN, 2), lambda nb: (nb, 0)),
        out_shape=jax.ShapeDtypeStruct((N, 2), f32),
    )(scat, embs_g, dinv, att.reshape(1, P),
      conv_z_W, czb, conv_r_W, crb, conv_h_W, chb,
      lin_z_W, lzb, lin_r_W, lrb, lin_h_W, lhb,
      cls_W1, b1, cls_W2, b2)

    return out.reshape(1, N, 2)
